# baseline (speedup 1.0000x reference)
import jax
import jax.numpy as jnp
from jax import lax
from jax.experimental import pallas as pl
from jax.experimental.pallas import tpu as pltpu

_VOCAB_PAD = 64
_H = 32
_HC = _H // 2
_GATES = 128
_S = 4
_CHUNK = 32


def _fused_call(feat, nodes1, adjf, adjb, tokw, whh, pad, w0, w1,
                *, B, L, BN, AW):
    NR = BN + 1
    NRp = ((NR + 7) // 8) * 8
    H = _H

    def _tree_sum(vals):
        while len(vals) > 1:
            nxt = [vals[i] + vals[i + 1] for i in range(0, len(vals) - 1, 2)]
            if len(vals) % 2:
                nxt.append(vals[-1])
            vals = nxt
        return vals[0]

    def _body(feat_ref, nodes_ref, adjf_ref, adjb_ref, tokw_ref, whh_ref,
              pad_ref, w0_ref, w1_ref,
              out_ref, hlast_ref, hid_ref, oh_s, xp_s, nr_s, s_s, a_s, h0e_s,
              whh_s):
        vlane = lax.broadcasted_iota(jnp.int32, (B, _VOCAB_PAD), 1)
        for t in range(L):
            fw = (feat_ref[0:B, t:t + 1] == vlane).astype(jnp.float32)
            bw = (feat_ref[0:B, L - 1 - t:L - t] == vlane).astype(jnp.float32)
            oh_s[pl.ds(t * B, B), :] = jnp.concatenate([fw, bw], axis=1)
        xp_s[...] = jnp.dot(oh_s[...], tokw_ref[...],
                            preferred_element_type=jnp.float32)

        for k in range(H):
            whh_s[pl.ds(k * B, B), :] = jnp.broadcast_to(
                whh_ref[k:k + 1, :], (B, _GATES))

        def _a_chunk(c):
            cw = min(_CHUNK, NR - c)
            colc = lax.broadcasted_iota(jnp.int32, (cw, NR), 1)
            acc_f = (adjf_ref[c:c + cw, 0:1] == colc).astype(jnp.float32)
            acc_b = (adjb_ref[c:c + cw, 0:1] == colc).astype(jnp.float32)
            for s in range(1, _S):
                acc_f = acc_f + (adjf_ref[c:c + cw, s:s + 1] == colc
                                 ).astype(jnp.float32)
                acc_b = acc_b + (adjb_ref[c:c + cw, s:s + 1] == colc
                                 ).astype(jnp.float32)
            a_s[pl.ds(c, cw), :] = acc_f.astype(jnp.bfloat16)
            a_s[pl.ds(NRp + c, cw), :] = acc_b.astype(jnp.bfloat16)

        def _s_chunk(c):
            cw = min(_CHUNK, BN - c)
            colc = lax.broadcasted_iota(jnp.int32, (cw, NR), 1)
            s_s[pl.ds(c, cw), :] = (
                nodes_ref[c:c + cw, :] == colc).astype(jnp.bfloat16)

        chunks = ([("a", c) for c in range(0, NR, _CHUNK)]
                  + [("s", c) for c in range(0, BN, _CHUNK)])

        h = jnp.zeros((B, H), jnp.float32)
        c_st = jnp.zeros((B, H), jnp.float32)
        hs = []
        for t in range(L):
            prods = [h[:, k:k + 1] * whh_s[pl.ds(k * B, B), :]
                     for k in range(H)]
            gates = _tree_sum(prods + [xp_s[pl.ds(t * B, B), :]])
            sig = jax.nn.sigmoid(gates[:, :3 * H])
            g_g = jnp.tanh(gates[:, 3 * H:])
            c_st = sig[:, H:2 * H] * c_st + sig[:, 0:H] * g_g
            h = sig[:, 2 * H:3 * H] * jnp.tanh(c_st)
            hs.append(h)
            if t < len(chunks):
                kind, c = chunks[t]
                (_a_chunk if kind == "a" else _s_chunk)(c)
        for kind, c in chunks[L:]:
            (_a_chunk if kind == "a" else _s_chunk)(c)
        fw_lane = lax.broadcasted_iota(jnp.int32, (B, H), 1) < _HC
        pairs = [jnp.where(fw_lane, hs[t], hs[L - 1 - t]) for t in range(L)]
        out_ref[...] = jnp.concatenate(pairs, axis=1)

        hl = hs[L - 1]
        rsel = lax.broadcasted_iota(jnp.int32, (B, B), 1)
        rmod = lax.broadcasted_iota(jnp.int32, (B, B), 0) % (B // 2)
        pa = (rsel == 2 * rmod).astype(jnp.float32)
        pb = (rsel == 2 * rmod + 1).astype(jnp.float32)
        av = jnp.dot(pa, hl, preferred_element_type=jnp.float32)
        bv = jnp.dot(pb, hl, preferred_element_type=jnp.float32)
        left = jnp.concatenate([av[0:B // 2, 0:_HC], av[B // 2:B, _HC:H]],
                               axis=0)
        right = jnp.concatenate([bv[0:B // 2, 0:_HC], bv[B // 2:B, _HC:H]],
                                axis=0)
        hlast_ref[...] = jnp.concatenate([left, right], axis=1)

        for t in range(L):
            nr_s[t:t + (B - 1) * L + 1:L, :] = pairs[t]
        nr_s[pl.ds(B * L, 1), :] = pad_ref[...]

        nr = nr_s[...]
        pres = jnp.sign(jnp.sum(jax.nn.relu(nr), axis=1, keepdims=True))
        nrp = jnp.concatenate(
            [nr, pres, jnp.zeros((NR, H - 1), jnp.float32)], axis=1)
        nrp_bf = nrp.astype(jnp.bfloat16)
        G = jnp.dot(a_s[...], nrp_bf,
                    preferred_element_type=jnp.float32)

        T = jnp.concatenate([nrp_bf, G[0:NR, :].astype(jnp.bfloat16),
                             G[NRp:NRp + NR, :].astype(jnp.bfloat16)],
                            axis=1)
        g = jnp.dot(s_s[...], T, preferred_element_type=jnp.float32)
        self_v = g[:, 0:H]
        sum_fw, nlen_fw = g[:, 64:64 + H], g[:, 64 + H:64 + H + 1]
        sum_bw, nlen_bw = g[:, 128:128 + H], g[:, 128 + H:128 + H + 1]
        inv_fw = pl.reciprocal(jnp.maximum(nlen_fw, 1.0), approx=False)
        inv_bw = pl.reciprocal(jnp.maximum(nlen_bw, 1.0), approx=False)

        x0 = jnp.concatenate([self_v, sum_fw * inv_fw, self_v, sum_bw * inv_bw],
                             axis=1)
        h0 = jax.nn.relu(jnp.dot(x0, w0_ref[...],
                                 preferred_element_type=jnp.float32))

        h0e_s[pl.ds(0, BN), :] = h0.astype(jnp.bfloat16)
        h0e_s[pl.ds(BN, NRp - BN), :] = jnp.zeros((NRp - BN, 4 * H),
                                                  jnp.bfloat16)
        M1 = jnp.dot(a_s[...], h0e_s[pl.ds(0, NR), :],
                     preferred_element_type=jnp.float32)
        T2 = jnp.concatenate(
            [M1[0:NR, 0:2 * H], M1[NRp:NRp + NR, 2 * H:4 * H]],
            axis=1).astype(jnp.bfloat16)
        m1g = jnp.dot(s_s[...], T2, preferred_element_type=jnp.float32)
        m1_fw = m1g[:, 0:2 * H] * inv_fw
        m1_bw = m1g[:, 2 * H:4 * H] * inv_bw
        x1 = jnp.concatenate([h0[:, 0:2 * H], m1_fw, h0[:, 2 * H:4 * H], m1_bw],
                             axis=1)
        graph = jax.nn.relu(jnp.dot(x1, w1_ref[...],
                                    preferred_element_type=jnp.float32))

        even = jnp.concatenate([graph[:, 0:H], graph[:, 2 * H:3 * H]], axis=1)
        odd = jnp.concatenate([graph[:, H:2 * H], graph[:, 3 * H:4 * H]], axis=1)
        hid_ref[0:2 * BN:2, :] = even
        hid_ref[1:2 * BN:2, :] = odd

    return pl.pallas_call(
        _body,
        out_shape=(jax.ShapeDtypeStruct((B, L * H), jnp.float32),
                   jax.ShapeDtypeStruct((B, H), jnp.float32),
                   jax.ShapeDtypeStruct((2 * BN, 2 * H), jnp.float32)),
        grid=(1,),
        in_specs=[
            pl.BlockSpec((B + 1, L), lambda i: (0, 0)),
            pl.BlockSpec((BN, 1), lambda i: (0, 0)),
            pl.BlockSpec((NR, AW), lambda i: (0, 0)),
            pl.BlockSpec((NR, AW), lambda i: (0, 0)),
            pl.BlockSpec((2 * _VOCAB_PAD, _GATES), lambda i: (0, 0)),
            pl.BlockSpec((_H, _GATES), lambda i: (0, 0)),
            pl.BlockSpec((1, _H), lambda i: (0, 0)),
            pl.BlockSpec((4 * _H, 4 * _H), lambda i: (0, 0)),
            pl.BlockSpec((8 * _H, 4 * _H), lambda i: (0, 0)),
        ],
        out_specs=(pl.BlockSpec((B, L * H), lambda i: (0, 0)),
                   pl.BlockSpec((B, H), lambda i: (0, 0)),
                   pl.BlockSpec((2 * BN, 2 * H), lambda i: (0, 0))),
        scratch_shapes=[pltpu.VMEM((L * B, _GATES), jnp.float32),
                        pltpu.VMEM((L * B, _GATES), jnp.float32),
                        pltpu.VMEM((NR, _H), jnp.float32),
                        pltpu.VMEM((BN, NR), jnp.bfloat16),
                        pltpu.VMEM((2 * NRp, NR), jnp.bfloat16),
                        pltpu.VMEM((NRp, 4 * _H), jnp.bfloat16),
                        pltpu.VMEM((_H * B, _GATES), jnp.float32)],
        compiler_params=pltpu.CompilerParams(dimension_semantics=("arbitrary",)),
    )(feat, nodes1, adjf, adjb, tokw, whh, pad, w0, w1)


def kernel(tokw, w_hh_fused, padding_vector, w0_big, w1_big,
           fw_adj_info, bw_adj_info, feature_info, batch_nodes, batch_wordlen):
    B, N = batch_nodes.shape
    H = _H
    L = feature_info.shape[1]

    nodes1 = batch_nodes.reshape(-1, 1).astype(jnp.int32)
    BN = nodes1.shape[0]
    AW = min(128, fw_adj_info.shape[1])

    out_flat, hlast2, hid = _fused_call(
        feature_info.astype(jnp.int32), nodes1, fw_adj_info.astype(jnp.int32),
        bw_adj_info.astype(jnp.int32), tokw, w_hh_fused, padding_vector,
        w0_big, w1_big, B=B, L=L, BN=BN, AW=AW)

    output_vector = out_flat.reshape(B, L, H)
    hidden = hid.reshape(-1, N, 2 * H)
    return output_vector, hlast2, hidden


# speedup vs baseline: 1.5720x; 1.5720x over previous
import jax
import jax.numpy as jnp
from jax import lax
from jax.experimental import pallas as pl
from jax.experimental.pallas import tpu as pltpu

_VOCAB_PAD = 64
_H = 32
_HC = _H // 2
_GATES = 128
_S = 4
_CHUNK = 32


def _fused_call(feat, nodes1, adjf, adjb, tokw, whh, pad, w0, w1,
                *, B, L, BN, AW):
    NR = BN + 1
    NRp = ((NR + 7) // 8) * 8
    H = _H

    def _body(feat_ref, nodes_ref, adjf_ref, adjb_ref, tokw_ref, whh_ref,
              pad_ref, w0_ref, w1_ref,
              out_ref, hlast_ref, hid_ref, oh_s, xp_s, nr_s, s_s, a_s, h0e_s,
              xp4_s):
        vlane = lax.broadcasted_iota(jnp.int32, (B, _VOCAB_PAD), 1)
        for t in range(L):
            fw = (feat_ref[0:B, t:t + 1] == vlane).astype(jnp.bfloat16)
            bw = (feat_ref[0:B, L - 1 - t:L - t] == vlane).astype(jnp.bfloat16)
            oh_s[pl.ds(t * B, B), :] = jnp.concatenate([fw, bw], axis=1)
        xp_s[...] = jnp.dot(oh_s[...], tokw_ref[...].astype(jnp.bfloat16),
                            preferred_element_type=jnp.float32)
        xp_v = xp_s[...]
        for q in range(4):
            xp4_s[q, :, :] = xp_v[:, q * H:(q + 1) * H]

        def _a_chunk(c):
            cw = min(_CHUNK, NR - c)
            colc = lax.broadcasted_iota(jnp.int32, (cw, NR), 1)
            acc_f = (adjf_ref[c:c + cw, 0:1] == colc).astype(jnp.float32)
            acc_b = (adjb_ref[c:c + cw, 0:1] == colc).astype(jnp.float32)
            for s in range(1, _S):
                acc_f = acc_f + (adjf_ref[c:c + cw, s:s + 1] == colc
                                 ).astype(jnp.float32)
                acc_b = acc_b + (adjb_ref[c:c + cw, s:s + 1] == colc
                                 ).astype(jnp.float32)
            a_s[pl.ds(c, cw), :] = acc_f.astype(jnp.bfloat16)
            a_s[pl.ds(NRp + c, cw), :] = acc_b.astype(jnp.bfloat16)

        def _s_chunk(c):
            cw = min(_CHUNK, BN - c)
            colc = lax.broadcasted_iota(jnp.int32, (cw, NR), 1)
            s_s[pl.ds(c, cw), :] = (
                nodes_ref[c:c + cw, :] == colc).astype(jnp.bfloat16)

        chunks = ([("a", c) for c in range(0, NR, _CHUNK)]
                  + [("s", c) for c in range(0, BN, _CHUNK)])

        h = jnp.zeros((B, H), jnp.float32)
        c_st = jnp.zeros((B, H), jnp.float32)
        hs = []
        whh_bf = whh_ref[...].astype(jnp.bfloat16)
        w_q = [whh_bf[:, q * H:(q + 1) * H] for q in range(4)]
        for t in range(L):
            hb = h.astype(jnp.bfloat16)
            g_i = xp4_s[0, pl.ds(t * B, B), :] + jnp.dot(
                hb, w_q[0], preferred_element_type=jnp.float32)
            g_f = xp4_s[1, pl.ds(t * B, B), :] + jnp.dot(
                hb, w_q[1], preferred_element_type=jnp.float32)
            g_o = xp4_s[2, pl.ds(t * B, B), :] + jnp.dot(
                hb, w_q[2], preferred_element_type=jnp.float32)
            g_g = xp4_s[3, pl.ds(t * B, B), :] + jnp.dot(
                hb, w_q[3], preferred_element_type=jnp.float32)
            c_st = (jax.nn.sigmoid(g_f) * c_st
                    + jax.nn.sigmoid(g_i) * jnp.tanh(g_g))
            h = jax.nn.sigmoid(g_o) * jnp.tanh(c_st)
            hs.append(h)
            if t < len(chunks):
                kind, c = chunks[t]
                (_a_chunk if kind == "a" else _s_chunk)(c)
        for kind, c in chunks[L:]:
            (_a_chunk if kind == "a" else _s_chunk)(c)
        fw_lane = lax.broadcasted_iota(jnp.int32, (B, H), 1) < _HC
        pairs = [jnp.where(fw_lane, hs[t], hs[L - 1 - t]) for t in range(L)]
        out_ref[...] = jnp.concatenate(pairs, axis=1)

        hl = hs[L - 1]
        rsel = lax.broadcasted_iota(jnp.int32, (B, B), 1)
        rmod = lax.broadcasted_iota(jnp.int32, (B, B), 0) % (B // 2)
        pa = (rsel == 2 * rmod).astype(jnp.float32)
        pb = (rsel == 2 * rmod + 1).astype(jnp.float32)
        av = jnp.dot(pa, hl, preferred_element_type=jnp.float32)
        bv = jnp.dot(pb, hl, preferred_element_type=jnp.float32)
        left = jnp.concatenate([av[0:B // 2, 0:_HC], av[B // 2:B, _HC:H]],
                               axis=0)
        right = jnp.concatenate([bv[0:B // 2, 0:_HC], bv[B // 2:B, _HC:H]],
                                axis=0)
        hlast_ref[...] = jnp.concatenate([left, right], axis=1)

        for t in range(L):
            nr_s[t:t + (B - 1) * L + 1:L, :] = pairs[t]
        nr_s[pl.ds(B * L, 1), :] = pad_ref[...]

        nr = nr_s[...]
        pres = jnp.sign(jnp.sum(jax.nn.relu(nr), axis=1, keepdims=True))
        nrp = jnp.concatenate(
            [nr, pres, jnp.zeros((NR, H - 1), jnp.float32)], axis=1)
        nrp_bf = nrp.astype(jnp.bfloat16)
        G = jnp.dot(a_s[...], nrp_bf,
                    preferred_element_type=jnp.float32)

        T = jnp.concatenate([nrp_bf, G[0:NR, :].astype(jnp.bfloat16),
                             G[NRp:NRp + NR, :].astype(jnp.bfloat16)],
                            axis=1)
        g = jnp.dot(s_s[...], T, preferred_element_type=jnp.float32)
        self_v = g[:, 0:H]
        sum_fw, nlen_fw = g[:, 64:64 + H], g[:, 64 + H:64 + H + 1]
        sum_bw, nlen_bw = g[:, 128:128 + H], g[:, 128 + H:128 + H + 1]
        inv_fw = pl.reciprocal(jnp.maximum(nlen_fw, 1.0), approx=False)
        inv_bw = pl.reciprocal(jnp.maximum(nlen_bw, 1.0), approx=False)

        x0 = jnp.concatenate([self_v, sum_fw * inv_fw, self_v, sum_bw * inv_bw],
                             axis=1)
        h0 = jax.nn.relu(jnp.dot(x0.astype(jnp.bfloat16),
                                 w0_ref[...].astype(jnp.bfloat16),
                                 preferred_element_type=jnp.float32))

        h0e_s[pl.ds(0, BN), :] = h0.astype(jnp.bfloat16)
        h0e_s[pl.ds(BN, NRp - BN), :] = jnp.zeros((NRp - BN, 4 * H),
                                                  jnp.bfloat16)
        M1 = jnp.dot(a_s[...], h0e_s[pl.ds(0, NR), :],
                     preferred_element_type=jnp.float32)
        T2 = jnp.concatenate(
            [M1[0:NR, 0:2 * H], M1[NRp:NRp + NR, 2 * H:4 * H]],
            axis=1).astype(jnp.bfloat16)
        m1g = jnp.dot(s_s[...], T2, preferred_element_type=jnp.float32)
        m1_fw = m1g[:, 0:2 * H] * inv_fw
        m1_bw = m1g[:, 2 * H:4 * H] * inv_bw
        x1 = jnp.concatenate([h0[:, 0:2 * H], m1_fw, h0[:, 2 * H:4 * H], m1_bw],
                             axis=1)
        graph = jax.nn.relu(jnp.dot(x1.astype(jnp.bfloat16),
                                    w1_ref[...].astype(jnp.bfloat16),
                                    preferred_element_type=jnp.float32))

        even = jnp.concatenate([graph[:, 0:H], graph[:, 2 * H:3 * H]], axis=1)
        odd = jnp.concatenate([graph[:, H:2 * H], graph[:, 3 * H:4 * H]], axis=1)
        hid_ref[0:2 * BN:2, :] = even
        hid_ref[1:2 * BN:2, :] = odd

    return pl.pallas_call(
        _body,
        out_shape=(jax.ShapeDtypeStruct((B, L * H), jnp.float32),
                   jax.ShapeDtypeStruct((B, H), jnp.float32),
                   jax.ShapeDtypeStruct((2 * BN, 2 * H), jnp.float32)),
        grid=(1,),
        in_specs=[
            pl.BlockSpec((B + 1, L), lambda i: (0, 0)),
            pl.BlockSpec((BN, 1), lambda i: (0, 0)),
            pl.BlockSpec((NR, AW), lambda i: (0, 0)),
            pl.BlockSpec((NR, AW), lambda i: (0, 0)),
            pl.BlockSpec((2 * _VOCAB_PAD, _GATES), lambda i: (0, 0)),
            pl.BlockSpec((_H, _GATES), lambda i: (0, 0)),
            pl.BlockSpec((1, _H), lambda i: (0, 0)),
            pl.BlockSpec((4 * _H, 4 * _H), lambda i: (0, 0)),
            pl.BlockSpec((8 * _H, 4 * _H), lambda i: (0, 0)),
        ],
        out_specs=(pl.BlockSpec((B, L * H), lambda i: (0, 0)),
                   pl.BlockSpec((B, H), lambda i: (0, 0)),
                   pl.BlockSpec((2 * BN, 2 * H), lambda i: (0, 0))),
        scratch_shapes=[pltpu.VMEM((L * B, _GATES), jnp.bfloat16),
                        pltpu.VMEM((L * B, _GATES), jnp.float32),
                        pltpu.VMEM((NR, _H), jnp.float32),
                        pltpu.VMEM((BN, NR), jnp.bfloat16),
                        pltpu.VMEM((2 * NRp, NR), jnp.bfloat16),
                        pltpu.VMEM((NRp, 4 * _H), jnp.bfloat16),
                        pltpu.VMEM((4, L * B, _H), jnp.float32)],
        compiler_params=pltpu.CompilerParams(dimension_semantics=("arbitrary",)),
    )(feat, nodes1, adjf, adjb, tokw, whh, pad, w0, w1)


def kernel(tokw, w_hh_fused, padding_vector, w0_big, w1_big,
           fw_adj_info, bw_adj_info, feature_info, batch_nodes, batch_wordlen):
    B, N = batch_nodes.shape
    H = _H
    L = feature_info.shape[1]

    nodes1 = batch_nodes.reshape(-1, 1).astype(jnp.int32)
    BN = nodes1.shape[0]
    AW = min(128, fw_adj_info.shape[1])

    out_flat, hlast2, hid = _fused_call(
        feature_info.astype(jnp.int32), nodes1, fw_adj_info.astype(jnp.int32),
        bw_adj_info.astype(jnp.int32), tokw, w_hh_fused, padding_vector,
        w0_big, w1_big, B=B, L=L, BN=BN, AW=AW)

    output_vector = out_flat.reshape(B, L, H)
    hidden = hid.reshape(-1, N, 2 * H)
    return output_vector, hlast2, hidden
